# Initial kernel scaffold; baseline (speedup 1.0000x reference)
import functools

import jax
import jax.numpy as jnp
from jax import lax
from jax.experimental import pallas as pl
from jax.experimental.pallas import tpu as pltpu
from jax.experimental.pallas import tpu_sc as plsc

N = 10000
D = 128
E = 320000

NUM_CORES = 2
NUM_SUBCORES = 16
NUM_WORKERS = NUM_CORES * NUM_SUBCORES
EDGES_PER_WORKER = E // NUM_WORKERS
CHUNK = 80
CHUNKS_PER_WORKER = EDGES_PER_WORKER // CHUNK
ROWS_PER_TILE = N // NUM_SUBCORES
WB = 125
WB_ITERS = ROWS_PER_TILE // WB


def _sc_agg_kernel(x_hbm, src_hbm, dst_hbm, out_hbm,
                   zbuf, src_v, dst_v, rows_v, agg_sh):
    c = lax.axis_index("c")
    s = lax.axis_index("s")
    wid = c * NUM_SUBCORES + s

    def zrow(i, _):
        for j in range(D // 16):
            zbuf[i, pl.ds(j * 16, 16)] = jnp.zeros((16,), jnp.float32)
        return 0
    lax.fori_loop(0, WB, zrow, 0)

    def zcopy(i, _):
        pltpu.sync_copy(zbuf, agg_sh.at[pl.ds(s * ROWS_PER_TILE + i * WB, WB)])
        return 0
    lax.fori_loop(0, WB_ITERS, zcopy, 0)

    plsc.subcore_barrier()

    base = wid * EDGES_PER_WORKER

    def ebody(i, _):
        off = base + i * CHUNK
        pltpu.sync_copy(src_hbm.at[pl.ds(off, CHUNK)], src_v)
        pltpu.sync_copy(dst_hbm.at[pl.ds(off, CHUNK)], dst_v)
        pltpu.sync_copy(x_hbm.at[src_v], rows_v)
        pltpu.sync_copy(rows_v, agg_sh.at[dst_v], add=True)
        return 0
    lax.fori_loop(0, CHUNKS_PER_WORKER, ebody, 0)

    plsc.subcore_barrier()

    def wb(i, _):
        sl = pl.ds(s * ROWS_PER_TILE + i * WB, WB)
        pltpu.sync_copy(agg_sh.at[sl], zbuf)
        pltpu.sync_copy(zbuf, out_hbm.at[c].at[sl])
        return 0
    lax.fori_loop(0, WB_ITERS, wb, 0)


def _sc_agg(h, src, dst):
    mesh = plsc.VectorSubcoreMesh(core_axis_name="c", subcore_axis_name="s")
    f = functools.partial(
        pl.kernel,
        mesh=mesh,
        out_type=jax.ShapeDtypeStruct((NUM_CORES, N, D), jnp.float32),
        scratch_types=[
            pltpu.VMEM((WB, D), jnp.float32),
            pltpu.VMEM((CHUNK,), jnp.int32),
            pltpu.VMEM((CHUNK,), jnp.int32),
            pltpu.VMEM((CHUNK, D), jnp.float32),
            pltpu.VMEM_SHARED((N, D), jnp.float32),
        ],
    )(_sc_agg_kernel)
    return f(h, src, dst)


def _tc_layer_kernel(h_ref, p0_ref, p1_ref, w_ref, b_ref, g_ref, bt_ref, o_ref):
    hs = h_ref[...] + p0_ref[...] + p1_ref[...]
    z = lax.dot_general(hs, w_ref[...], (((1,), (1,)), ((), ())),
                        preferred_element_type=jnp.float32) + b_ref[...]
    r = jnp.maximum(z, 0.0)
    mu = jnp.mean(r, axis=0, keepdims=True)
    var = jnp.mean((r - mu) ** 2, axis=0, keepdims=True)
    o_ref[...] = (r - mu) * lax.rsqrt(var + 1e-5) * g_ref[...] + bt_ref[...]


def _tc_layer(h, p0, p1, w, b, g, bt):
    return pl.pallas_call(
        _tc_layer_kernel,
        out_shape=jax.ShapeDtypeStruct((N, D), jnp.float32),
    )(h, p0, p1, w, b.reshape(1, D), g.reshape(1, D), bt.reshape(1, D))


def _tc_final_kernel(h_ref, p0_ref, p1_ref, w_ref, b_ref, o_ref):
    hs = h_ref[...] + p0_ref[...] + p1_ref[...]
    z = lax.dot_general(hs, w_ref[...], (((1,), (1,)), ((), ())),
                        preferred_element_type=jnp.float32) + b_ref[...]
    m = jnp.max(z, axis=1, keepdims=True)
    e = jnp.exp(z - m)
    lse = jnp.log(jnp.sum(e, axis=1, keepdims=True)) + m
    o_ref[...] = z - lse


def _tc_final(h, p0, p1, w, b):
    c = w.shape[0]
    return pl.pallas_call(
        _tc_final_kernel,
        out_shape=jax.ShapeDtypeStruct((N, c), jnp.float32),
    )(h, p0, p1, w, b.reshape(1, c))


def kernel(x, edge_index, proj_W, proj_b, W0, b0, W1, b1, W2, b2,
           final_W, final_b, norm_g, norm_b,
           g0, bt0, g1, bt1, g2, bt2):
    src = edge_index[0]
    dst = edge_index[1]

    h = x
    p = _sc_agg(h, src, dst)
    h = _tc_layer(h, p[0], p[1], proj_W, proj_b, norm_g, norm_b)
    for w, b, g, bt in ((W0, b0, g0, bt0), (W1, b1, g1, bt1),
                        (W2, b2, g2, bt2)):
        p = _sc_agg(h, src, dst)
        h = _tc_layer(h, p[0], p[1], w, b, g, bt)
    p = _sc_agg(h, src, dst)
    return _tc_final(h, p[0], p[1], final_W, final_b)


# speedup vs baseline: 4.3994x; 4.3994x over previous
import functools

import jax
import jax.numpy as jnp
from jax import lax
from jax.experimental import pallas as pl
from jax.experimental.pallas import tpu as pltpu
from jax.experimental.pallas import tpu_sc as plsc

N = 10000
D = 128
E = 320000

NUM_CORES = 2
NUM_SUBCORES = 16
NUM_WORKERS = NUM_CORES * NUM_SUBCORES
EDGES_PER_WORKER = E // NUM_WORKERS
CHUNK = 80
CHUNKS_PER_WORKER = EDGES_PER_WORKER // CHUNK
N_PAD = 10240
ROWS_PER_TILE = N_PAD // NUM_SUBCORES
WB = 128
WB_ITERS = ROWS_PER_TILE // WB


def _sc_agg_kernel(x_hbm, src_hbm, dst_hbm, out_hbm,
                   zbuf, src_v, dst_v, rows_v, agg_sh):
    c = lax.axis_index("c")
    s = lax.axis_index("s")
    wid = c * NUM_SUBCORES + s

    def zrow(i, _):
        for j in range(D // 16):
            zbuf[i, pl.ds(j * 16, 16)] = jnp.zeros((16,), jnp.float32)
        return 0
    lax.fori_loop(0, WB, zrow, 0)

    def zcopy(i, _):
        pltpu.sync_copy(zbuf, agg_sh.at[pl.ds(s * ROWS_PER_TILE + i * WB, WB)])
        return 0
    lax.fori_loop(0, WB_ITERS, zcopy, 0)

    plsc.subcore_barrier()

    base = wid * EDGES_PER_WORKER

    def ebody(i, _):
        off = base + i * CHUNK
        pltpu.sync_copy(src_hbm.at[pl.ds(off, CHUNK)], src_v)
        pltpu.sync_copy(dst_hbm.at[pl.ds(off, CHUNK)], dst_v)
        pltpu.sync_copy(x_hbm.at[src_v], rows_v)
        pltpu.sync_copy(rows_v, agg_sh.at[dst_v], add=True)
        return 0
    lax.fori_loop(0, CHUNKS_PER_WORKER, ebody, 0)

    plsc.subcore_barrier()

    def wb(i, _):
        sl = pl.ds(s * ROWS_PER_TILE + i * WB, WB)
        pltpu.sync_copy(agg_sh.at[sl], zbuf)
        pltpu.sync_copy(zbuf, out_hbm.at[c].at[sl])
        return 0
    lax.fori_loop(0, WB_ITERS, wb, 0)


def _sc_agg(h, src, dst):
    mesh = plsc.VectorSubcoreMesh(core_axis_name="c", subcore_axis_name="s")
    f = functools.partial(
        pl.kernel,
        mesh=mesh,
        out_type=jax.ShapeDtypeStruct((NUM_CORES, N_PAD, D), jnp.float32),
        scratch_types=[
            pltpu.VMEM((WB, D), jnp.float32),
            pltpu.VMEM((CHUNK,), jnp.int32),
            pltpu.VMEM((CHUNK,), jnp.int32),
            pltpu.VMEM((CHUNK, D), jnp.float32),
            pltpu.VMEM_SHARED((N_PAD, D), jnp.float32),
        ],
    )(_sc_agg_kernel)
    p = f(h, src, dst)
    return p[0, :N], p[1, :N]


def _tc_layer_kernel(h_ref, p0_ref, p1_ref, w_ref, b_ref, g_ref, bt_ref, o_ref):
    hs = h_ref[...] + p0_ref[...] + p1_ref[...]
    z = lax.dot_general(hs, w_ref[...], (((1,), (1,)), ((), ())),
                        preferred_element_type=jnp.float32) + b_ref[...]
    r = jnp.maximum(z, 0.0)
    mu = jnp.mean(r, axis=0, keepdims=True)
    var = jnp.mean((r - mu) ** 2, axis=0, keepdims=True)
    o_ref[...] = (r - mu) * lax.rsqrt(var + 1e-5) * g_ref[...] + bt_ref[...]


def _tc_layer(h, p0, p1, w, b, g, bt):
    return pl.pallas_call(
        _tc_layer_kernel,
        out_shape=jax.ShapeDtypeStruct((N, D), jnp.float32),
    )(h, p0, p1, w, b.reshape(1, D), g.reshape(1, D), bt.reshape(1, D))


def _tc_final_kernel(h_ref, p0_ref, p1_ref, w_ref, b_ref, o_ref):
    hs = h_ref[...] + p0_ref[...] + p1_ref[...]
    z = lax.dot_general(hs, w_ref[...], (((1,), (1,)), ((), ())),
                        preferred_element_type=jnp.float32) + b_ref[...]
    m = jnp.max(z, axis=1, keepdims=True)
    e = jnp.exp(z - m)
    lse = jnp.log(jnp.sum(e, axis=1, keepdims=True)) + m
    o_ref[...] = z - lse


def _tc_final(h, p0, p1, w, b):
    c = w.shape[0]
    return pl.pallas_call(
        _tc_final_kernel,
        out_shape=jax.ShapeDtypeStruct((N, c), jnp.float32),
    )(h, p0, p1, w, b.reshape(1, c))


def kernel(x, edge_index, proj_W, proj_b, W0, b0, W1, b1, W2, b2,
           final_W, final_b, norm_g, norm_b,
           g0, bt0, g1, bt1, g2, bt2):
    src = edge_index[0]
    dst = edge_index[1]

    h = x
    p0, p1 = _sc_agg(h, src, dst)
    h = _tc_layer(h, p0, p1, proj_W, proj_b, norm_g, norm_b)
    for w, b, g, bt in ((W0, b0, g0, bt0), (W1, b1, g1, bt1),
                        (W2, b2, g2, bt2)):
        p0, p1 = _sc_agg(h, src, dst)
        h = _tc_layer(h, p0, p1, w, b, g, bt)
    p0, p1 = _sc_agg(h, src, dst)
    return _tc_final(h, p0, p1, final_W, final_b)
